# baseline (speedup 1.0000x reference)
import functools

import jax
import jax.numpy as jnp
from jax import lax
from jax.experimental import pallas as pl
from jax.experimental.pallas import tpu as pltpu
from jax.experimental.pallas import tpu_sc as plsc

B, N, D = 2, 2048, 2048
E, K, H, OUT = 8, 2, 1024, 2048
TAU = 1.5
T = B * N
TM = 256
NTOT = K * T + E * TM
NT = NTOT // TM
TB = 512

NW = 32
RPW = NTOT // NW
GCH = 16
RPT = T // NW


def _router_body(h_ref, wr_ref, br_ref, meta_ref, psum_ref, lsum_ref,
                 csum_ref, esum_ref):
    i = pl.program_id(0)

    @pl.when(i == 0)
    def _init():
        psum_ref[...] = jnp.zeros_like(psum_ref)
        lsum_ref[...] = jnp.zeros_like(lsum_ref)
        csum_ref[...] = jnp.zeros_like(csum_ref)
        esum_ref[...] = jnp.zeros_like(esum_ref)

    x = h_ref[...]
    logits = jnp.dot(x, wr_ref[...], preferred_element_type=jnp.float32)
    logits = (logits + br_ref[...]) / TAU

    cols = lax.broadcasted_iota(jnp.int32, (TB, E), 1)
    m1 = jnp.max(logits, axis=1, keepdims=True)
    i1 = jnp.min(jnp.where(logits == m1, cols, E), axis=1, keepdims=True)
    oh1 = cols == i1
    l2 = jnp.where(oh1, -jnp.inf, logits)
    m2 = jnp.max(l2, axis=1, keepdims=True)
    i2 = jnp.min(jnp.where(l2 == m2, cols, E), axis=1, keepdims=True)
    oh2 = cols == i2

    t = jnp.exp(m2 - m1)
    s2 = 1.0 + t
    g1 = 1.0 / s2
    g2 = t / s2

    p = jnp.exp(logits - m1)
    s8 = jnp.sum(p, axis=1, keepdims=True)
    probs = p / s8
    logp = (logits - m1) - jnp.log(s8)
    ent_col = jnp.sum(-probs * logp, axis=0, keepdims=True)

    occ = oh1.astype(jnp.float32) + oh2.astype(jnp.float32)
    ii = lax.broadcasted_iota(jnp.int32, (TB, TB), 0)
    jj = lax.broadcasted_iota(jnp.int32, (TB, TB), 1)
    tril = (jj < ii).astype(jnp.float32)
    cum_excl = jnp.dot(tril, occ, preferred_element_type=jnp.float32)
    carry = csum_ref[...]
    rank_base = carry + cum_excl
    r1 = jnp.sum(jnp.where(oh1, rank_base, 0.0), axis=1, keepdims=True)
    r2 = jnp.sum(jnp.where(oh2, rank_base, 0.0), axis=1, keepdims=True)

    zz = jnp.zeros_like(r1)
    meta_ref[...] = jnp.concatenate(
        [i1.astype(jnp.float32), i2.astype(jnp.float32), g1, g2, r1, r2,
         zz, zz], axis=1)
    csum_ref[...] = carry + jnp.sum(occ, axis=0, keepdims=True)
    lsum_ref[...] += jnp.sum(jnp.where(oh1, g1, 0.0) +
                             jnp.where(oh2, g2, 0.0), axis=0, keepdims=True)
    psum_ref[...] += jnp.sum(probs, axis=0, keepdims=True)
    esum_ref[...] += ent_col


def _router(h_flat, Wr, br2):
    spec_acc = pl.BlockSpec((1, E), lambda i: (0, 0))
    return pl.pallas_call(
        _router_body,
        grid=(T // TB,),
        in_specs=[
            pl.BlockSpec((TB, D), lambda i: (i, 0)),
            pl.BlockSpec((D, E), lambda i: (0, 0)),
            pl.BlockSpec((1, E), lambda i: (0, 0)),
        ],
        out_specs=[pl.BlockSpec((TB, E), lambda i: (i, 0)),
                   spec_acc, spec_acc, spec_acc, spec_acc],
        out_shape=[
            jax.ShapeDtypeStruct((T, E), jnp.float32),
            jax.ShapeDtypeStruct((1, E), jnp.float32),
            jax.ShapeDtypeStruct((1, E), jnp.float32),
            jax.ShapeDtypeStruct((1, E), jnp.float32),
            jax.ShapeDtypeStruct((1, E), jnp.float32),
        ],
    )(h_flat, Wr, br2)


def _sc_mesh():
    return plsc.VectorSubcoreMesh(core_axis_name="c", subcore_axis_name="s")


@functools.partial(
    pl.kernel,
    out_type=(jax.ShapeDtypeStruct((NTOT, D), jnp.float32),
              jax.ShapeDtypeStruct((NTOT,), jnp.float32)),
    mesh=_sc_mesh(),
    scratch_types=[
        pltpu.VMEM((T,), jnp.int32),
        pltpu.VMEM((T,), jnp.int32),
        pltpu.VMEM((T,), jnp.float32),
        pltpu.VMEM((T,), jnp.float32),
        pltpu.VMEM((NTOT,), jnp.int32),
        pltpu.VMEM((NTOT,), jnp.float32),
        pltpu.VMEM((GCH, D), jnp.float32),
        pltpu.SemaphoreType.DMA,
    ],
    compiler_params=pltpu.CompilerParams(needs_layout_passes=False),
)
def _sc_gather(pos1_hbm, pos2_hbm, g1_hbm, g2_hbm, h_hbm, x_hbm, gs_hbm,
               pos1_v, pos2_v, g1_v, g2_v, ids_v, gl_v, buf_v, sem):
    wid = lax.axis_index("s") * 2 + lax.axis_index("c")
    base = wid * RPW

    pltpu.sync_copy(pos1_hbm, pos1_v)
    pltpu.sync_copy(pos2_hbm, pos2_v)
    pltpu.sync_copy(g1_hbm, g1_v)
    pltpu.sync_copy(g2_hbm, g2_v)

    zi = jnp.zeros((16,), jnp.int32)
    zf = jnp.zeros((16,), jnp.float32)

    def _zinit(j, c):
        ids_v[pl.ds(j * 16, 16)] = zi
        gl_v[pl.ds(j * 16, 16)] = zf
        return c

    lax.fori_loop(0, NTOT // 16, _zinit, 0)

    lane = lax.iota(jnp.int32, 16)

    def _scat(j, c):
        o = j * 16
        tok = lane + o
        p1 = pos1_v[pl.ds(o, 16)]
        p2 = pos2_v[pl.ds(o, 16)]
        plsc.store_scatter(ids_v, [p1], tok)
        plsc.store_scatter(ids_v, [p2], tok)
        plsc.store_scatter(gl_v, [p1], g1_v[pl.ds(o, 16)])
        plsc.store_scatter(gl_v, [p2], g2_v[pl.ds(o, 16)])
        return c

    lax.fori_loop(0, T // 16, _scat, 0)

    pltpu.sync_copy(gl_v.at[pl.ds(base, RPW)], gs_hbm.at[pl.ds(base, RPW)])

    def _grow(j, c):
        o = base + j * GCH
        idx = ids_v[pl.ds(o, GCH)]
        pltpu.async_copy(h_hbm.at[idx], buf_v, sem).wait()
        pltpu.sync_copy(buf_v, x_hbm.at[pl.ds(o, GCH)])
        return c

    lax.fori_loop(0, RPW // GCH, _grow, 0)


def _ffn_body(expert_ref, valid_ref, x_ref, g_ref, w1_ref, b1_ref, w2_ref,
              b2_ref, y_ref):
    i = pl.program_id(0)

    @pl.when(valid_ref[i] == 1)
    def _():
        x = x_ref[...]
        a = jnp.dot(x, w1_ref[0], preferred_element_type=jnp.float32)
        a = a + b1_ref[0]
        a = jax.nn.silu(a)
        g = g_ref[...]
        a = a * g
        y = jnp.dot(a, w2_ref[0], preferred_element_type=jnp.float32)
        y_ref[...] = y + g * b2_ref[0]


def _ffn(tile_expert, tile_valid, x_sorted, g_sorted, W1, b1, W2, b2):
    grid_spec = pltpu.PrefetchScalarGridSpec(
        num_scalar_prefetch=2,
        grid=(NT,),
        in_specs=[
            pl.BlockSpec((TM, D), lambda i, e, v: (i, 0)),
            pl.BlockSpec((TM, 1), lambda i, e, v: (i, 0)),
            pl.BlockSpec((1, D, H), lambda i, e, v: (e[i], 0, 0)),
            pl.BlockSpec((1, 1, H), lambda i, e, v: (e[i], 0, 0)),
            pl.BlockSpec((1, H, OUT), lambda i, e, v: (e[i], 0, 0)),
            pl.BlockSpec((1, 1, OUT), lambda i, e, v: (e[i], 0, 0)),
        ],
        out_specs=pl.BlockSpec((TM, OUT), lambda i, e, v: (i, 0)),
    )
    return pl.pallas_call(
        _ffn_body,
        grid_spec=grid_spec,
        out_shape=jax.ShapeDtypeStruct((NTOT, OUT), jnp.float32),
    )(tile_expert, tile_valid, x_sorted, g_sorted,
      W1, b1.reshape(E, 1, H), W2, b2.reshape(E, 1, OUT))


@functools.partial(
    pl.kernel,
    out_type=jax.ShapeDtypeStruct((T, OUT), jnp.float32),
    mesh=_sc_mesh(),
    scratch_types=[
        pltpu.VMEM((RPT,), jnp.int32),
        pltpu.VMEM((RPT,), jnp.int32),
        pltpu.VMEM((GCH, OUT), jnp.float32),
        pltpu.VMEM((GCH, OUT), jnp.float32),
        pltpu.SemaphoreType.DMA,
        pltpu.SemaphoreType.DMA,
    ],
    compiler_params=pltpu.CompilerParams(needs_layout_passes=False),
)
def _sc_combine(pos1_hbm, pos2_hbm, y_hbm, out_hbm, p1_v, p2_v, bufa, bufb,
                sema, semb):
    wid = lax.axis_index("s") * 2 + lax.axis_index("c")
    base = wid * RPT

    pltpu.sync_copy(pos1_hbm.at[pl.ds(base, RPT)], p1_v)
    pltpu.sync_copy(pos2_hbm.at[pl.ds(base, RPT)], p2_v)

    def _chunk(cc, c):
        o = cc * GCH
        ia = p1_v[pl.ds(o, 16)]
        ib = p2_v[pl.ds(o, 16)]
        cpa = pltpu.async_copy(y_hbm.at[ia], bufa, sema)
        cpb = pltpu.async_copy(y_hbm.at[ib], bufb, semb)
        cpa.wait()
        cpb.wait()
        for r in range(GCH):
            def _add(j, c2):
                oo = j * 64
                for u in range(4):
                    sl = pl.ds(oo + u * 16, 16)
                    bufa[r, sl] = bufa[r, sl] + bufb[r, sl]
                return c2
            lax.fori_loop(0, OUT // 64, _add, 0)
        pltpu.sync_copy(bufa, out_hbm.at[pl.ds(base + o, GCH)])
        return c

    lax.fori_loop(0, RPT // GCH, _chunk, 0)


def kernel(h, Wr, br, W1, b1, W2, b2):
    h_flat = h.reshape(T, D)
    meta, psum, lsum, csum, esum = _router(h_flat, Wr, br.reshape(1, E))

    counts = csum[0].astype(jnp.int32)
    nt_e = (counts + TM - 1) // TM
    cum_tiles = jnp.cumsum(nt_e)
    row_off = (cum_tiles - nt_e) * TM
    total_tiles = cum_tiles[E - 1]
    tj = jnp.arange(NT, dtype=jnp.int32)
    tile_expert = jnp.minimum(
        jnp.sum((tj[:, None] >= cum_tiles[None, :]).astype(jnp.int32), axis=1),
        E - 1).astype(jnp.int32)
    tile_valid = (tj < total_tiles).astype(jnp.int32)

    i1 = meta[:, 0].astype(jnp.int32)
    i2 = meta[:, 1].astype(jnp.int32)
    g1 = meta[:, 2]
    g2 = meta[:, 3]
    pos1 = jnp.take(row_off, i1) + meta[:, 4].astype(jnp.int32)
    pos2 = jnp.take(row_off, i2) + meta[:, 5].astype(jnp.int32)

    x_sorted, g_sorted = _sc_gather(pos1, pos2, g1, g2, h_flat)
    y_sorted = _ffn(tile_expert, tile_valid, x_sorted,
                    g_sorted.reshape(NTOT, 1), W1, b1, W2, b2)
    out = _sc_combine(pos1, pos2, y_sorted)

    y = out.reshape(B, N, OUT)
    p_e = psum[0] / T
    f_e = lsum[0] / T
    entropy = jnp.sum(esum) / T
    aux = jnp.maximum(E * jnp.sum(p_e * f_e) - 1.0, 0.0)
    return (y, aux, lax.stop_gradient(p_e), lax.stop_gradient(f_e),
            lax.stop_gradient(entropy))


# speedup vs baseline: 1.0206x; 1.0206x over previous
import functools

import jax
import jax.numpy as jnp
from jax import lax
from jax.experimental import pallas as pl
from jax.experimental.pallas import tpu as pltpu
from jax.experimental.pallas import tpu_sc as plsc

B, N, D = 2, 2048, 2048
E, K, H, OUT = 8, 2, 1024, 2048
TAU = 1.5
T = B * N
TM = 256
NTOT = K * T + E * TM
NT = NTOT // TM
TB = 512

NW = 32
RPW = NTOT // NW
GCH = 16
RPT = T // NW


def _router_body(h_ref, wr_ref, br_ref, meta_ref, psum_ref, lsum_ref,
                 csum_ref, esum_ref):
    i = pl.program_id(0)

    @pl.when(i == 0)
    def _init():
        psum_ref[...] = jnp.zeros_like(psum_ref)
        lsum_ref[...] = jnp.zeros_like(lsum_ref)
        csum_ref[...] = jnp.zeros_like(csum_ref)
        esum_ref[...] = jnp.zeros_like(esum_ref)

    x = h_ref[...]
    logits = jnp.dot(x, wr_ref[...], preferred_element_type=jnp.float32)
    logits = (logits + br_ref[...]) / TAU

    cols = lax.broadcasted_iota(jnp.int32, (TB, E), 1)
    m1 = jnp.max(logits, axis=1, keepdims=True)
    i1 = jnp.min(jnp.where(logits == m1, cols, E), axis=1, keepdims=True)
    oh1 = cols == i1
    l2 = jnp.where(oh1, -jnp.inf, logits)
    m2 = jnp.max(l2, axis=1, keepdims=True)
    i2 = jnp.min(jnp.where(l2 == m2, cols, E), axis=1, keepdims=True)
    oh2 = cols == i2

    t = jnp.exp(m2 - m1)
    s2 = 1.0 + t
    g1 = 1.0 / s2
    g2 = t / s2

    p = jnp.exp(logits - m1)
    s8 = jnp.sum(p, axis=1, keepdims=True)
    probs = p / s8
    logp = (logits - m1) - jnp.log(s8)
    ent_col = jnp.sum(-probs * logp, axis=0, keepdims=True)

    occ = oh1.astype(jnp.float32) + oh2.astype(jnp.float32)
    ii = lax.broadcasted_iota(jnp.int32, (TB, TB), 0)
    jj = lax.broadcasted_iota(jnp.int32, (TB, TB), 1)
    tril = (jj < ii).astype(jnp.float32)
    cum_excl = jnp.dot(tril, occ, preferred_element_type=jnp.float32)
    carry = csum_ref[...]
    rank_base = carry + cum_excl
    r1 = jnp.sum(jnp.where(oh1, rank_base, 0.0), axis=1, keepdims=True)
    r2 = jnp.sum(jnp.where(oh2, rank_base, 0.0), axis=1, keepdims=True)

    zz = jnp.zeros_like(r1)
    meta_ref[...] = jnp.concatenate(
        [i1.astype(jnp.float32), i2.astype(jnp.float32), g1, g2, r1, r2,
         zz, zz], axis=1)
    csum_ref[...] = carry + jnp.sum(occ, axis=0, keepdims=True)
    lsum_ref[...] += jnp.sum(jnp.where(oh1, g1, 0.0) +
                             jnp.where(oh2, g2, 0.0), axis=0, keepdims=True)
    psum_ref[...] += jnp.sum(probs, axis=0, keepdims=True)
    esum_ref[...] += ent_col


def _router(h_flat, Wr, br2):
    spec_acc = pl.BlockSpec((1, E), lambda i: (0, 0))
    return pl.pallas_call(
        _router_body,
        grid=(T // TB,),
        in_specs=[
            pl.BlockSpec((TB, D), lambda i: (i, 0)),
            pl.BlockSpec((D, E), lambda i: (0, 0)),
            pl.BlockSpec((1, E), lambda i: (0, 0)),
        ],
        out_specs=[pl.BlockSpec((TB, E), lambda i: (i, 0)),
                   spec_acc, spec_acc, spec_acc, spec_acc],
        out_shape=[
            jax.ShapeDtypeStruct((T, E), jnp.float32),
            jax.ShapeDtypeStruct((1, E), jnp.float32),
            jax.ShapeDtypeStruct((1, E), jnp.float32),
            jax.ShapeDtypeStruct((1, E), jnp.float32),
            jax.ShapeDtypeStruct((1, E), jnp.float32),
        ],
    )(h_flat, Wr, br2)


def _sc_mesh():
    return plsc.VectorSubcoreMesh(core_axis_name="c", subcore_axis_name="s")


@functools.partial(
    pl.kernel,
    out_type=(jax.ShapeDtypeStruct((NTOT, D), jnp.float32),
              jax.ShapeDtypeStruct((NTOT,), jnp.float32)),
    mesh=_sc_mesh(),
    scratch_types=[
        pltpu.VMEM((T,), jnp.int32),
        pltpu.VMEM((T,), jnp.int32),
        pltpu.VMEM((T,), jnp.float32),
        pltpu.VMEM((T,), jnp.float32),
        pltpu.VMEM((NTOT,), jnp.int32),
        pltpu.VMEM((NTOT,), jnp.float32),
        pltpu.VMEM((GCH, D), jnp.float32),
        pltpu.VMEM((GCH, D), jnp.float32),
        pltpu.SemaphoreType.DMA,
        pltpu.SemaphoreType.DMA,
        pltpu.SemaphoreType.DMA,
        pltpu.SemaphoreType.DMA,
    ],
    compiler_params=pltpu.CompilerParams(needs_layout_passes=False),
)
def _sc_gather(pos1_hbm, pos2_hbm, g1_hbm, g2_hbm, h_hbm, x_hbm, gs_hbm,
               pos1_v, pos2_v, g1_v, g2_v, ids_v, gl_v, buf0, buf1,
               si0, si1, so0, so1):
    wid = lax.axis_index("s") * 2 + lax.axis_index("c")
    base = wid * RPW

    pltpu.sync_copy(pos1_hbm, pos1_v)
    pltpu.sync_copy(pos2_hbm, pos2_v)
    pltpu.sync_copy(g1_hbm, g1_v)
    pltpu.sync_copy(g2_hbm, g2_v)

    zi = jnp.zeros((16,), jnp.int32)
    zf = jnp.zeros((16,), jnp.float32)

    def _zinit(j, c):
        o = base + j * 16
        ids_v[pl.ds(o, 16)] = zi
        gl_v[pl.ds(o, 16)] = zf
        return c

    lax.fori_loop(0, RPW // 16, _zinit, 0)

    lane = lax.iota(jnp.int32, 16)

    def _scat(j, c):
        o = j * 16
        tok = lane + o
        p1 = pos1_v[pl.ds(o, 16)]
        p2 = pos2_v[pl.ds(o, 16)]
        plsc.store_scatter(ids_v, [p1], tok)
        plsc.store_scatter(ids_v, [p2], tok)
        plsc.store_scatter(gl_v, [p1], g1_v[pl.ds(o, 16)])
        plsc.store_scatter(gl_v, [p2], g2_v[pl.ds(o, 16)])
        return c

    lax.fori_loop(0, T // 16, _scat, 0)

    pltpu.sync_copy(gl_v.at[pl.ds(base, RPW)], gs_hbm.at[pl.ds(base, RPW)])

    bufs = (buf0, buf1)
    sin = (si0, si1)
    sout = (so0, so1)
    nch = RPW // GCH
    ins = [None] * nch
    outs = [None] * nch

    def _start_in(c):
        idx = ids_v[pl.ds(base + c * GCH, GCH)]
        ins[c] = pltpu.async_copy(h_hbm.at[idx], bufs[c % 2], sin[c % 2])

    _start_in(0)
    for c in range(nch):
        b = c % 2
        if c + 1 < nch:
            if c >= 1:
                outs[c - 1].wait()
            _start_in(c + 1)
        ins[c].wait()
        outs[c] = pltpu.async_copy(
            bufs[b], x_hbm.at[pl.ds(base + c * GCH, GCH)], sout[b])
    outs[nch - 2].wait()
    outs[nch - 1].wait()


def _ffn_body(expert_ref, valid_ref, x_ref, g_ref, w1_ref, b1_ref, w2_ref,
              b2_ref, y_ref):
    i = pl.program_id(0)

    @pl.when(valid_ref[i] == 1)
    def _():
        x = x_ref[...]
        a = jnp.dot(x, w1_ref[0], preferred_element_type=jnp.float32)
        a = a + b1_ref[0]
        a = jax.nn.silu(a)
        g = g_ref[...]
        a = a * g
        y = jnp.dot(a, w2_ref[0], preferred_element_type=jnp.float32)
        y_ref[...] = y + g * b2_ref[0]


def _ffn(tile_expert, tile_valid, x_sorted, g_sorted, W1, b1, W2, b2):
    grid_spec = pltpu.PrefetchScalarGridSpec(
        num_scalar_prefetch=2,
        grid=(NT,),
        in_specs=[
            pl.BlockSpec((TM, D), lambda i, e, v: (i, 0)),
            pl.BlockSpec((TM, 1), lambda i, e, v: (i, 0)),
            pl.BlockSpec((1, D, H), lambda i, e, v: (e[i], 0, 0)),
            pl.BlockSpec((1, 1, H), lambda i, e, v: (e[i], 0, 0)),
            pl.BlockSpec((1, H, OUT), lambda i, e, v: (e[i], 0, 0)),
            pl.BlockSpec((1, 1, OUT), lambda i, e, v: (e[i], 0, 0)),
        ],
        out_specs=pl.BlockSpec((TM, OUT), lambda i, e, v: (i, 0)),
    )
    return pl.pallas_call(
        _ffn_body,
        grid_spec=grid_spec,
        out_shape=jax.ShapeDtypeStruct((NTOT, OUT), jnp.float32),
    )(tile_expert, tile_valid, x_sorted, g_sorted,
      W1, b1.reshape(E, 1, H), W2, b2.reshape(E, 1, OUT))


CCH = 8


@functools.partial(
    pl.kernel,
    out_type=jax.ShapeDtypeStruct((T, OUT), jnp.float32),
    mesh=_sc_mesh(),
    scratch_types=[
        pltpu.VMEM((RPT,), jnp.int32),
        pltpu.VMEM((RPT,), jnp.int32),
        pltpu.VMEM((CCH, OUT), jnp.float32),
        pltpu.VMEM((CCH, OUT), jnp.float32),
        pltpu.VMEM((CCH, OUT), jnp.float32),
        pltpu.VMEM((CCH, OUT), jnp.float32),
        pltpu.SemaphoreType.DMA,
        pltpu.SemaphoreType.DMA,
        pltpu.SemaphoreType.DMA,
        pltpu.SemaphoreType.DMA,
        pltpu.SemaphoreType.DMA,
        pltpu.SemaphoreType.DMA,
    ],
    compiler_params=pltpu.CompilerParams(needs_layout_passes=False),
)
def _sc_combine(pos1_hbm, pos2_hbm, y_hbm, out_hbm, p1_v, p2_v,
                bufa0, bufa1, bufb0, bufb1, sa0, sa1, sb0, sb1, so0, so1):
    wid = lax.axis_index("s") * 2 + lax.axis_index("c")
    base = wid * RPT

    pltpu.sync_copy(pos1_hbm.at[pl.ds(base, RPT)], p1_v)
    pltpu.sync_copy(pos2_hbm.at[pl.ds(base, RPT)], p2_v)

    bufa = (bufa0, bufa1)
    bufb = (bufb0, bufb1)
    sa = (sa0, sa1)
    sb = (sb0, sb1)
    so = (so0, so1)
    nch = RPT // CCH
    ins_a = [None] * nch
    ins_b = [None] * nch
    outs = [None] * nch

    def _start_in(c):
        b = c % 2
        o = c * CCH
        ins_a[c] = pltpu.async_copy(
            y_hbm.at[p1_v.at[pl.ds(o, CCH)]], bufa[b], sa[b])
        ins_b[c] = pltpu.async_copy(
            y_hbm.at[p2_v.at[pl.ds(o, CCH)]], bufb[b], sb[b])

    _start_in(0)
    for c in range(nch):
        b = c % 2
        if c + 1 < nch:
            if c >= 1:
                outs[c - 1].wait()
            _start_in(c + 1)
        ins_a[c].wait()
        ins_b[c].wait()
        for r in range(CCH):
            def _add(j, c2, _r=r, _b=b):
                oo = j * 64
                for u in range(4):
                    sl = pl.ds(oo + u * 16, 16)
                    bufa[_b][_r, sl] = bufa[_b][_r, sl] + bufb[_b][_r, sl]
                return c2
            lax.fori_loop(0, OUT // 64, _add, 0)
        outs[c] = pltpu.async_copy(
            bufa[b], out_hbm.at[pl.ds(base + c * CCH, CCH)], so[b])
    outs[nch - 2].wait()
    outs[nch - 1].wait()


def kernel(h, Wr, br, W1, b1, W2, b2):
    h_flat = h.reshape(T, D)
    meta, psum, lsum, csum, esum = _router(h_flat, Wr, br.reshape(1, E))

    counts = csum[0].astype(jnp.int32)
    nt_e = (counts + TM - 1) // TM
    cum_tiles = jnp.cumsum(nt_e)
    row_off = (cum_tiles - nt_e) * TM
    total_tiles = cum_tiles[E - 1]
    tj = jnp.arange(NT, dtype=jnp.int32)
    tile_expert = jnp.minimum(
        jnp.sum((tj[:, None] >= cum_tiles[None, :]).astype(jnp.int32), axis=1),
        E - 1).astype(jnp.int32)
    tile_valid = (tj < total_tiles).astype(jnp.int32)

    i1 = meta[:, 0].astype(jnp.int32)
    i2 = meta[:, 1].astype(jnp.int32)
    g1 = meta[:, 2]
    g2 = meta[:, 3]
    pos1 = jnp.take(row_off, i1) + meta[:, 4].astype(jnp.int32)
    pos2 = jnp.take(row_off, i2) + meta[:, 5].astype(jnp.int32)

    x_sorted, g_sorted = _sc_gather(pos1, pos2, g1, g2, h_flat)
    y_sorted = _ffn(tile_expert, tile_valid, x_sorted,
                    g_sorted.reshape(NTOT, 1), W1, b1, W2, b2)
    out = _sc_combine(pos1, pos2, y_sorted)

    y = out.reshape(B, N, OUT)
    p_e = psum[0] / T
    f_e = lsum[0] / T
    entropy = jnp.sum(esum) / T
    aux = jnp.maximum(E * jnp.sum(p_e * f_e) - 1.0, 0.0)
    return (y, aux, lax.stop_gradient(p_e), lax.stop_gradient(f_e),
            lax.stop_gradient(entropy))
